# Initial kernel scaffold; baseline (speedup 1.0000x reference)
import jax
import jax.numpy as jnp
from jax.experimental import pallas as pl


def kernel(x, edge_index, edge_attr, params):
    raise NotImplementedError("write your pallas kernel here")


# speedup vs baseline: 5.3039x; 5.3039x over previous
import jax
import jax.numpy as jnp
from jax.experimental import pallas as pl
from jax.experimental.pallas import tpu as pltpu

_N = 10
_NP = 16
_E = 90
_EP = 128
_TILE = 256
_CI = (512, 1024, 1024, 2048, 2048, 256)
_CO = (1024, 1024, 2048, 2048, 256, 256)
_NT = tuple(c // _TILE for c in _CO)
_START = (0, 4, 8, 16, 24, 25)
_STEPS = 26
_PREC = jax.lax.Precision.HIGHEST
_PREC_REF = jax.lax.Precision.DEFAULT


def _attention_matrix(sacc, dacc, e_ref):
    ones_row = jnp.ones((1, _TILE), jnp.float32)
    asrc_row = jax.lax.dot_general(
        ones_row, sacc[...], (((1,), (1,)), ((), ())),
        preferred_element_type=jnp.float32, precision=_PREC)
    adst_col = jnp.sum(dacc[...], axis=1, keepdims=True)
    a = adst_col + asrc_row
    a = jnp.where(a > 0, a, 0.2 * a)
    srow = e_ref[0:1, :]
    drow = e_ref[1:2, :]
    niota = jax.lax.broadcasted_iota(jnp.int32, (_NP, _EP), 0)
    oh_s = (srow == niota).astype(jnp.float32)
    oh_d = (drow == niota).astype(jnp.float32)
    cnt = jax.lax.dot_general(
        oh_d, oh_s, (((1,), (1,)), ((), ())),
        preferred_element_type=jnp.float32, precision=_PREC)
    r = jax.lax.broadcasted_iota(jnp.int32, (_NP, _NP), 0)
    c = jax.lax.broadcasted_iota(jnp.int32, (_NP, _NP), 1)
    cnt = cnt + ((r == c) & (r < _N)).astype(jnp.float32)
    has = cnt > 0
    am = jnp.max(jnp.where(has, a, -1e30), axis=1, keepdims=True)
    has_any = jnp.sum(cnt, axis=1, keepdims=True) > 0
    am = jnp.where(has_any, am, 0.0)
    ex = jnp.where(has, jnp.exp(a - am), 0.0) * cnt
    den = jnp.sum(ex, axis=1, keepdims=True)
    return ex / (den + 1e-16)


def _body(x_ref, e_ref, w0, w1, w2, w3, w4, w5, tbl_ref, heads_ref,
          out_ref, xb, hb, sacc, dacc, bb):
    t = pl.program_id(0)
    w_refs = (w0, w1, w2, w3, w4, w5)
    for i in range(6):
        @pl.when((t >= _START[i]) & (t < _START[i] + _NT[i]))
        def _(i=i):
            j = t - _START[i]
            if i == 0:
                xin = x_ref[...]
            else:
                xin = xb[:, : _CI[i]]
            h = jax.lax.dot_general(
                xin, w_refs[i][...], (((1,), (1,)), ((), ())),
                preferred_element_type=jnp.float32, precision=_PREC_REF)
            tbl = tbl_ref[0]
            first = j == 0
            sc = h * tbl[0:1, :]
            dc = h * tbl[1:2, :]
            sacc[...] = jnp.where(first, sc, sacc[...] + sc)
            dacc[...] = jnp.where(first, dc, dacc[...] + dc)
            hb[:, pl.ds(j * _TILE, _TILE)] = h
            bb[0:1, pl.ds(j * _TILE, _TILE)] = tbl[2:3, :]

            @pl.when(t == _START[i] + _NT[i] - 1)
            def _():
                att = _attention_matrix(sacc, dacc, e_ref)
                hfull = hb[:, : _CO[i]]
                out = jax.lax.dot_general(
                    att, hfull, (((1,), (0,)), ((), ())),
                    preferred_element_type=jnp.float32, precision=_PREC)
                out = jnp.maximum(out + bb[0:1, : _CO[i]], 0.0)
                xb[:, : _CO[i]] = out
                if i == 5:
                    lin_row = heads_ref[0:1, :_NP]
                    z = jax.lax.dot_general(
                        lin_row, out, (((1,), (0,)), ((), ())),
                        preferred_element_type=jnp.float32,
                        precision=_PREC_REF) + heads_ref[3:4, 0:1]
                    def _bf(u):
                        return u.astype(jnp.bfloat16).astype(jnp.float32)
                    v = (jnp.sum(_bf(jnp.tanh(z)) * _bf(heads_ref[1:2, :]),
                                 axis=1, keepdims=True)
                         + heads_ref[3:4, 1:2])
                    cc = (jnp.sum(_bf(jnp.maximum(z, 0.0))
                                  * _bf(heads_ref[2:3, :]),
                                  axis=1, keepdims=True)
                          + heads_ref[3:4, 2:3])
                    cc = jax.nn.sigmoid(cc)
                    ri = jax.lax.broadcasted_iota(jnp.int32, (8, 128), 0)
                    ci = jax.lax.broadcasted_iota(jnp.int32, (8, 128), 1)
                    res = jnp.where((ri == 0) & (ci == 0), v, 0.0)
                    res = res + jnp.where((ri == 0) & (ci == 1), cc, 0.0)
                    out_ref[...] = res


def kernel(x, edge_index, edge_attr, params):
    del edge_attr
    p = params
    names = ('g1a', 'g1b', 'g2a', 'g2b', 'g3a', 'g3b')
    ws = [p[n + '_W'] for n in names]
    a_s = jnp.concatenate([p[n + '_as'] for n in names]).reshape(_STEPS, 1, _TILE)
    a_d = jnp.concatenate([p[n + '_ad'] for n in names]).reshape(_STEPS, 1, _TILE)
    b = jnp.concatenate([p[n + '_b'] for n in names]).reshape(_STEPS, 1, _TILE)
    tbl = jnp.concatenate([a_s, a_d, b], axis=1)
    lin_row = jnp.concatenate(
        [p['lin_W'][0], jnp.zeros((_TILE - _N,), jnp.float32)])[None, :]
    scal = jnp.concatenate(
        [p['lin_b'], p['reg_b'], p['cls_b'],
         jnp.zeros((_TILE - 3,), jnp.float32)])[None, :]
    heads = jnp.concatenate([lin_row, p['reg_W'], p['cls_W'], scal], axis=0)
    xp = jnp.pad(x, ((0, _NP - _N), (0, 0)))
    ep = jnp.pad(edge_index, ((0, 0), (0, _EP - _E)), constant_values=-1)

    in_specs = [
        pl.BlockSpec((_NP, _CI[0]), lambda t: (0, 0)),
        pl.BlockSpec((2, _EP), lambda t: (0, 0)),
    ]
    for i in range(6):
        in_specs.append(pl.BlockSpec(
            (_TILE, _CI[i]),
            lambda t, i=i: (jnp.clip(t - _START[i], 0, _NT[i] - 1), 0)))
    in_specs.append(pl.BlockSpec((1, 3, _TILE), lambda t: (t, 0, 0)))
    in_specs.append(pl.BlockSpec((4, _TILE), lambda t: (0, 0)))

    out = pl.pallas_call(
        _body,
        grid=(_STEPS,),
        in_specs=in_specs,
        out_specs=pl.BlockSpec((8, 128), lambda t: (0, 0)),
        out_shape=jax.ShapeDtypeStruct((8, 128), jnp.float32),
        scratch_shapes=[
            pltpu.VMEM((_NP, 2048), jnp.float32),
            pltpu.VMEM((_NP, 2048), jnp.float32),
            pltpu.VMEM((_NP, _TILE), jnp.float32),
            pltpu.VMEM((_NP, _TILE), jnp.float32),
            pltpu.VMEM((8, 2048), jnp.float32),
        ],
        compiler_params=pltpu.CompilerParams(
            dimension_semantics=("arbitrary",)),
    )(xp, ep, *ws, tbl, heads)
    return (out[0, 0:1], out[0, 1:2])


# speedup vs baseline: 6.5109x; 1.2276x over previous
import jax
import jax.numpy as jnp
from jax.experimental import pallas as pl
from jax.experimental.pallas import tpu as pltpu

_N = 10
_NP = 16
_E = 90
_EP = 128
_TILE = 256
_CI = (512, 1024, 1024, 2048, 2048, 256)
_CO = (1024, 1024, 2048, 2048, 256, 256)
_NT = tuple(c // _TILE for c in _CO)
_NS = (2, 2, 4, 4)
_S2 = (0, 2, 4, 8)
_STEPS = 13
_PREC = jax.lax.Precision.HIGHEST
_PREC_REF = jax.lax.Precision.DEFAULT


def _attention_matrix(sacc, dacc, e_ref):
    ones_row = jnp.ones((1, _TILE), jnp.float32)
    asrc_row = jax.lax.dot_general(
        ones_row, sacc, (((1,), (1,)), ((), ())),
        preferred_element_type=jnp.float32, precision=_PREC)
    adst_col = jnp.sum(dacc, axis=1, keepdims=True)
    a = adst_col + asrc_row
    a = jnp.where(a > 0, a, 0.2 * a)
    srow = e_ref[0:1, :]
    drow = e_ref[1:2, :]
    niota = jax.lax.broadcasted_iota(jnp.int32, (_NP, _EP), 0)
    oh_s = (srow == niota).astype(jnp.float32)
    oh_d = (drow == niota).astype(jnp.float32)
    cnt = jax.lax.dot_general(
        oh_d, oh_s, (((1,), (1,)), ((), ())),
        preferred_element_type=jnp.float32, precision=_PREC)
    r = jax.lax.broadcasted_iota(jnp.int32, (_NP, _NP), 0)
    c = jax.lax.broadcasted_iota(jnp.int32, (_NP, _NP), 1)
    cnt = cnt + ((r == c) & (r < _N)).astype(jnp.float32)
    has = cnt > 0
    am = jnp.max(jnp.where(has, a, -1e30), axis=1, keepdims=True)
    has_any = jnp.sum(cnt, axis=1, keepdims=True) > 0
    am = jnp.where(has_any, am, 0.0)
    ex = jnp.where(has, jnp.exp(a - am), 0.0) * cnt
    den = jnp.sum(ex, axis=1, keepdims=True)
    return ex / (den + 1e-16)


def _gat_tail(h, a_s, a_d, b_row, e_ref):
    att = _attention_matrix(h * a_s, h * a_d, e_ref)
    out = jax.lax.dot_general(
        att, h, (((1,), (0,)), ((), ())),
        preferred_element_type=jnp.float32, precision=_PREC)
    return jnp.maximum(out + b_row, 0.0)


def _body(x_ref, e_ref, wa0, wb0, wa1, wb1, wa2, wb2, wa3, wb3, w4, w5,
          tbl_ref, heads_ref, out_ref, xb, hbuf, sacc, dacc, bb):
    t = pl.program_id(0)
    tbl = tbl_ref[0]
    wa = (wa0, wa1, wa2, wa3)
    wb = (wb0, wb1, wb2, wb3)
    for i in range(4):
        @pl.when((t >= _S2[i]) & (t < _S2[i] + _NS[i]))
        def _(i=i):
            j = t - _S2[i]
            if i == 0:
                xin = x_ref[...]
            else:
                xin = xb[:, : _CI[i]]
            ha = jax.lax.dot_general(
                xin, wa[i][0], (((1,), (1,)), ((), ())),
                preferred_element_type=jnp.float32, precision=_PREC_REF)
            hc = jax.lax.dot_general(
                xin, wb[i][0], (((1,), (1,)), ((), ())),
                preferred_element_type=jnp.float32, precision=_PREC_REF)
            sc = ha * tbl[0:1, :] + hc * tbl[3:4, :]
            dc = ha * tbl[1:2, :] + hc * tbl[4:5, :]
            first = j == 0
            sacc[...] = jnp.where(first, sc, sacc[...] + sc)
            dacc[...] = jnp.where(first, dc, dacc[...] + dc)
            hbuf[:, pl.ds((2 * j) * _TILE, _TILE)] = ha
            hbuf[:, pl.ds((2 * j + 1) * _TILE, _TILE)] = hc
            bb[0:1, pl.ds((2 * j) * _TILE, _TILE)] = tbl[2:3, :]
            bb[0:1, pl.ds((2 * j + 1) * _TILE, _TILE)] = tbl[5:6, :]

            @pl.when(t == _S2[i] + _NS[i] - 1)
            def _():
                att = _attention_matrix(sacc[...], dacc[...], e_ref)
                hfull = hbuf[:, : _CO[i]]
                out = jax.lax.dot_general(
                    att, hfull, (((1,), (0,)), ((), ())),
                    preferred_element_type=jnp.float32, precision=_PREC)
                out = jnp.maximum(out + bb[0:1, : _CO[i]], 0.0)
                xb[:, : _CO[i]] = out

    @pl.when(t == _STEPS - 1)
    def _():
        h4 = jax.lax.dot_general(
            xb[:, : _CI[4]], w4[...], (((1,), (1,)), ((), ())),
            preferred_element_type=jnp.float32, precision=_PREC_REF)
        out4 = _gat_tail(h4, tbl[0:1, :], tbl[1:2, :], tbl[2:3, :], e_ref)
        h5 = jax.lax.dot_general(
            out4, w5[...], (((1,), (1,)), ((), ())),
            preferred_element_type=jnp.float32, precision=_PREC_REF)
        out5 = _gat_tail(h5, tbl[3:4, :], tbl[4:5, :], tbl[5:6, :], e_ref)
        lin_row = heads_ref[0:1, :_NP]
        z = jax.lax.dot_general(
            lin_row, out5, (((1,), (0,)), ((), ())),
            preferred_element_type=jnp.float32,
            precision=_PREC_REF) + heads_ref[3:4, 0:1]

        def _bf(u):
            return u.astype(jnp.bfloat16).astype(jnp.float32)
        v = (jnp.sum(_bf(jnp.tanh(z)) * _bf(heads_ref[1:2, :]),
                     axis=1, keepdims=True)
             + heads_ref[3:4, 1:2])
        cc = (jnp.sum(_bf(jnp.maximum(z, 0.0)) * _bf(heads_ref[2:3, :]),
                      axis=1, keepdims=True)
              + heads_ref[3:4, 2:3])
        cc = jax.nn.sigmoid(cc)
        ri = jax.lax.broadcasted_iota(jnp.int32, (8, 128), 0)
        ci = jax.lax.broadcasted_iota(jnp.int32, (8, 128), 1)
        res = jnp.where((ri == 0) & (ci == 0), v, 0.0)
        res = res + jnp.where((ri == 0) & (ci == 1), cc, 0.0)
        out_ref[...] = res


def kernel(x, edge_index, edge_attr, params):
    del edge_attr
    p = params
    names = ('g1a', 'g1b', 'g2a', 'g2b', 'g3a', 'g3b')
    ws = [p[n + '_W'].reshape(_NT[i], _TILE, _CI[i])
          for i, n in enumerate(names)]

    rows = []
    for i, n in enumerate(names[:4]):
        a_s = p[n + '_as'].reshape(_NS[i], 2, _TILE)
        a_d = p[n + '_ad'].reshape(_NS[i], 2, _TILE)
        b = p[n + '_b'].reshape(_NS[i], 2, _TILE)
        rows.append(jnp.stack(
            [a_s[:, 0], a_d[:, 0], b[:, 0], a_s[:, 1], a_d[:, 1], b[:, 1]],
            axis=1))
    rows.append(jnp.stack(
        [p['g3a_as'], p['g3a_ad'], p['g3a_b'],
         p['g3b_as'], p['g3b_ad'], p['g3b_b']], axis=0)[None])
    tbl = jnp.concatenate(rows, axis=0)

    lin_row = jnp.concatenate(
        [p['lin_W'][0], jnp.zeros((_TILE - _N,), jnp.float32)])[None, :]
    scal = jnp.concatenate(
        [p['lin_b'], p['reg_b'], p['cls_b'],
         jnp.zeros((_TILE - 3,), jnp.float32)])[None, :]
    heads = jnp.concatenate([lin_row, p['reg_W'], p['cls_W'], scal], axis=0)
    xp = jnp.pad(x, ((0, _NP - _N), (0, 0)))
    ep = jnp.pad(edge_index, ((0, 0), (0, _EP - _E)), constant_values=-1)

    in_specs = [
        pl.BlockSpec((_NP, _CI[0]), lambda t: (0, 0)),
        pl.BlockSpec((2, _EP), lambda t: (0, 0)),
    ]
    for i in range(4):
        in_specs.append(pl.BlockSpec(
            (1, _TILE, _CI[i]),
            lambda t, i=i: (jnp.clip(2 * (t - _S2[i]), 0, _NT[i] - 2), 0, 0)))
        in_specs.append(pl.BlockSpec(
            (1, _TILE, _CI[i]),
            lambda t, i=i: (jnp.clip(2 * (t - _S2[i]) + 1, 1, _NT[i] - 1),
                            0, 0)))
    in_specs.append(pl.BlockSpec((_TILE, _CI[4]), lambda t: (0, 0)))
    in_specs.append(pl.BlockSpec((_TILE, _CI[5]), lambda t: (0, 0)))
    in_specs.append(pl.BlockSpec((1, 6, _TILE), lambda t: (t, 0, 0)))
    in_specs.append(pl.BlockSpec((4, _TILE), lambda t: (0, 0)))

    w_args = []
    for i in range(4):
        w_args += [ws[i], ws[i]]
    w_args += [p['g3a_W'], p['g3b_W']]

    out = pl.pallas_call(
        _body,
        grid=(_STEPS,),
        in_specs=in_specs,
        out_specs=pl.BlockSpec((8, 128), lambda t: (0, 0)),
        out_shape=jax.ShapeDtypeStruct((8, 128), jnp.float32),
        scratch_shapes=[
            pltpu.VMEM((_NP, 2048), jnp.float32),
            pltpu.VMEM((_NP, 2048), jnp.float32),
            pltpu.VMEM((_NP, _TILE), jnp.float32),
            pltpu.VMEM((_NP, _TILE), jnp.float32),
            pltpu.VMEM((8, 2048), jnp.float32),
        ],
        compiler_params=pltpu.CompilerParams(
            dimension_semantics=("arbitrary",)),
    )(xp, ep, *w_args, tbl, heads)
    return (out[0, 0:1], out[0, 1:2])
